# baseline (speedup 1.0000x reference)
import functools

import jax
import jax.numpy as jnp
import numpy as np
from jax import lax
from jax.experimental import pallas as pl
from jax.experimental.pallas import tpu as pltpu
from jax.experimental.pallas import tpu_sc as plsc

_N_LEVELS = 16
_TABLE_SIZE = 2 ** 19
_MASK = _TABLE_SIZE - 1
_BASE_RES = 16
_MAX_RES = 4096
_SCALE = np.exp((np.log(_MAX_RES) - np.log(_BASE_RES)) / (_N_LEVELS - 1))
_RES = [int(np.floor(_BASE_RES * _SCALE ** l)) for l in range(_N_LEVELS)]
_P1 = np.uint32(2654435761).astype(np.int32)
_P2 = np.int32(805459861)

_NW = 32
_C = 128
_LANES = 16


def _vperm(v, idx):
    return lax.gather(
        v, idx[:, None],
        lax.GatherDimensionNumbers(offset_dims=(), collapsed_slice_dims=(0,),
                                   start_index_map=(0,)),
        slice_sizes=(1,), mode=lax.GatherScatterMode.PROMISE_IN_BOUNDS)


def _enc_body(xyz_hbm, resf_hbm, table_hbm, out_hbm,
              pos_v, acc_v, resf_v, idx_v, w_v, rows_v, spm_v,
              sem0, sem1, sem2):
    n = out_hbm.shape[0] // (2 * _N_LEVELS)
    b = n // _NW
    nch = b // _C
    sid = lax.axis_index("s")
    wid = sid * 2 + lax.axis_index("c")
    base = wid * b
    sl = _TABLE_SIZE // 16
    for k in range(3):
        pltpu.sync_copy(xyz_hbm.at[pl.ds(k * n + base, b)], pos_v.at[k])
    pltpu.sync_copy(resf_hbm, resf_v)
    resvec = resf_v[pl.ds(0, _LANES)]
    sems = (sem0, sem1)

    def stage(l):
        pltpu.async_copy(
            table_hbm.at[pl.ds(l * _TABLE_SIZE + sid * sl, sl)],
            spm_v.at[l & 1, pl.ds(sid * sl, sl)], sem2)

    def stage_wait():
        pltpu.make_async_copy(table_hbm.at[pl.ds(0, sl)],
                              spm_v.at[0, pl.ds(0, sl)], sem2).wait()

    stage(0)
    stage_wait()
    plsc.subcore_barrier()

    def level_body(l, _):
        hs = _vperm(resvec, jnp.broadcast_to(l, (_LANES,))) * 0.5
        arow = (l & 3) * 2
        tsrc = spm_v.at[l & 1]

        def idxgen_fire(c, buf):
            for i in range(_C // _LANES):
                o = c * _C + i * _LANES
                px = pos_v[0, pl.ds(o, _LANES)]
                py = pos_v[1, pl.ds(o, _LANES)]
                pz = pos_v[2, pl.ds(o, _LANES)]
                posx = px * hs + hs
                posy = py * hs + hs
                posz = pz * hs + hs
                ix = posx.astype(jnp.int32)
                iy = posy.astype(jnp.int32)
                iz = posz.astype(jnp.int32)
                fx = posx - ix.astype(jnp.float32)
                fy = posy - iy.astype(jnp.float32)
                fz = posz - iz.astype(jnp.float32)
                hx = (ix, ix + 1)
                t1 = iy * _P1
                t2 = iz * _P2
                hy = (t1, t1 + _P1)
                hz = (t2, t2 + _P2)
                wx = (1.0 - fx, fx)
                wy = (1.0 - fy, fy)
                wz = (1.0 - fz, fz)
                wyz = {(dy, dz): wy[dy] * wz[dz]
                       for dy in (0, 1) for dz in (0, 1)}
                for dx in (0, 1):
                    for dy in (0, 1):
                        for dz in (0, 1):
                            ci = dx * 4 + dy * 2 + dz
                            idx_v[buf,
                                  pl.ds(ci * _C + i * _LANES, _LANES)] = (
                                ((hx[dx] ^ hy[dy]) ^ hz[dz]) & _MASK)
                            w_v[buf, ci, pl.ds(i * _LANES, _LANES)] = (
                                wx[dx] * wyz[(dy, dz)])
            pltpu.async_copy(tsrc.at[idx_v.at[buf]],
                             rows_v.at[buf], sems[buf])

        def drain(buf):
            pltpu.make_async_copy(tsrc.at[idx_v.at[buf]],
                                  rows_v.at[buf], sems[buf]).wait()

        def accum(c, buf):
            for i in range(_C // _LANES):
                acc = None
                for ci in range(8):
                    u = rows_v[buf, pl.ds(ci * _C + i * _LANES, _LANES)]
                    r = plsc.bitcast(u, jnp.bfloat16)
                    wv = w_v[buf, ci, pl.ds(i * _LANES, _LANES)]
                    wd = plsc.pack(wv, wv,
                                   format=plsc.PackFormat.INTERLEAVED)
                    acc = r * wd if acc is None else acc + r * wd
                f0a, f1a = plsc.unpack(acc,
                                       format=plsc.PackFormat.INTERLEAVED)
                o = c * (8 * _C) + arow * 128 + i * _LANES
                acc_v[pl.ds(o, _LANES)] = f0a
                acc_v[pl.ds(o + 128, _LANES)] = f1a

        idxgen_fire(0, 0)
        stage(jnp.minimum(l + 1, _N_LEVELS - 1))

        def pair(p, _):
            cc = p * 2
            idxgen_fire(cc + 1, 1)
            drain(0)
            accum(cc, 0)

            @pl.when(cc + 2 < nch)
            def _():
                idxgen_fire(cc + 2, 0)

            drain(1)
            accum(cc + 1, 1)
            return ()

        lax.fori_loop(0, nch // 2, pair, (), unroll=False)

        @pl.when((l & 3) == 3)
        def _():
            g = lax.shift_right_logical(l, 2)
            pltpu.sync_copy(acc_v,
                            out_hbm.at[pl.ds(g * (8 * n) + base * 8, 8 * b)])
        stage_wait()
        plsc.subcore_barrier()
        return ()

    lax.fori_loop(0, _N_LEVELS, level_body, (), unroll=False)


def _make_encode(n, interpret=False):
    b = n // _NW
    mesh = plsc.VectorSubcoreMesh(core_axis_name="c", subcore_axis_name="s",
                                  num_cores=2, num_subcores=16)
    return functools.partial(
        pl.kernel, _enc_body,
        out_type=jax.ShapeDtypeStruct((2 * _N_LEVELS * n,), jnp.float32),
        mesh=mesh,
        compiler_params=pltpu.CompilerParams(needs_layout_passes=False,
                                             use_tc_tiling_on_sc=False),
        scratch_types=[
            pltpu.VMEM((3, b), jnp.float32),
            pltpu.VMEM((8 * b,), jnp.float32),
            pltpu.VMEM((128,), jnp.float32),
            pltpu.VMEM((2, 8 * _C), jnp.int32),
            pltpu.VMEM((2, 8, _C), jnp.float32),
            pltpu.VMEM((2, 8 * _C), jnp.uint32),
            pltpu.VMEM_SHARED((2, _TABLE_SIZE), jnp.uint32),
            pltpu.SemaphoreType.DMA,
            pltpu.SemaphoreType.DMA,
            pltpu.SemaphoreType.DMA,
        ],
        interpret=interpret,
    )()


def _pack_body(x_ref, o_ref):
    x = x_ref[...]
    u = lax.bitcast_convert_type(x, jnp.uint32)
    rnd = (u + jnp.uint32(0x7FFF) + ((u >> 16) & jnp.uint32(1))) >> 16
    k = o_ref.shape[0]
    r3 = rnd.reshape(k, 2, 128)
    o_ref[...] = r3[:, 0, :] | (r3[:, 1, :] << 16)


def _make_pack(rows, bk=4096):
    grid = rows // bk
    return pl.pallas_call(
        _pack_body,
        grid=(grid,),
        in_specs=[pl.BlockSpec((bk, 128), lambda i: (i, 0))],
        out_specs=pl.BlockSpec((bk // 2, 128), lambda i: (i, 0)),
        out_shape=jax.ShapeDtypeStruct((rows // 2, 128), jnp.uint32),
    )


def _mm(a, bm):
    return lax.dot_general(a, bm, (((1,), (0,)), ((), ())),
                           preferred_element_type=jnp.float32)


def _mlp_body(enc_ref, dir_ref, w1_ref, w2_ref, h1_ref, h2_ref, h3_ref,
              rgb_ref, den_ref):
    e = enc_ref[...]
    h = jnp.maximum(_mm(w1_ref[...], e), 0.0)
    bo = _mm(w2_ref[...], h)
    den_ref[...] = jnp.exp(bo[0:1, :] - 1.0)
    geo = bo[1:, :]

    d = dir_ref[...]
    dd = ((d + 1.0) * 0.5) * 2.0 - 1.0
    x, y, z = dd[0:1, :], dd[1:2, :], dd[2:3, :]
    x2, y2, z2 = x * x, y * y, z * z
    xy, yz, xz = x * y, y * z, x * z
    sh = [
        jnp.full_like(x, 0.28209479177387814),
        -0.48860251190291987 * y,
        0.48860251190291987 * z,
        -0.48860251190291987 * x,
        1.0925484305920792 * xy,
        -1.0925484305920792 * yz,
        0.94617469575755997 * z2 - 0.31539156525252005,
        -1.0925484305920792 * xz,
        0.54627421529603959 * (x2 - y2),
        0.59004358992664352 * y * (-3.0 * x2 + y2),
        2.8906114426405538 * xy * z,
        0.45704579946446572 * y * (1.0 - 5.0 * z2),
        0.3731763325901154 * z * (5.0 * z2 - 3.0),
        0.45704579946446572 * x * (1.0 - 5.0 * z2),
        1.4453057213202769 * z * (x2 - y2),
        0.59004358992664352 * x * (-x2 + 3.0 * y2),
    ]
    h1 = h1_ref[...]
    hh = _mm(h1[:, 16:], geo)
    for j in range(16):
        hh = hh + h1[:, j:j + 1] * sh[j]
    hh = jnp.maximum(hh, 0.0)
    hh = jnp.maximum(_mm(h2_ref[...], hh), 0.0)
    rgb_ref[...] = jax.nn.sigmoid(_mm(h3_ref[...], hh))


def _make_mlp(n, bn=4096, interpret=False):
    grid = n // bn
    full = lambda shape: pl.BlockSpec(shape, lambda i: (0, 0))
    return pl.pallas_call(
        _mlp_body,
        grid=(grid,),
        in_specs=[
            pl.BlockSpec((2 * _N_LEVELS, bn), lambda i: (0, i)),
            pl.BlockSpec((3, bn), lambda i: (0, i)),
            full((64, 2 * _N_LEVELS)),
            full((16, 64)),
            full((64, 31)),
            full((64, 64)),
            full((3, 64)),
        ],
        out_specs=[
            pl.BlockSpec((3, bn), lambda i: (0, i)),
            pl.BlockSpec((1, bn), lambda i: (0, i)),
        ],
        out_shape=[
            jax.ShapeDtypeStruct((3, n), jnp.float32),
            jax.ShapeDtypeStruct((1, n), jnp.float32),
        ],
        interpret=interpret,
    )


def kernel(positions, directions, hash_table,
           w_base1, w_base2, w_head1, w_head2, w_head3):
    n = positions.shape[0]
    xyz = positions.T.reshape(-1)
    nrows = _N_LEVELS * (_TABLE_SIZE // 128) * 2
    table_view = (hash_table
                  .reshape(_N_LEVELS, _TABLE_SIZE // 128, 128, 2)
                  .transpose(0, 1, 3, 2)
                  .reshape(nrows, 128))
    table_flat = _make_pack(nrows)(table_view).reshape(-1)
    resf = jnp.asarray(np.array(_RES + [0] * (128 - _N_LEVELS),
                                dtype=np.float32))
    nsplit = 4
    nh = n // nsplit
    dir_t = directions.T
    wts = (w_base1.T, w_base2.T, w_head1.T, w_head2.T, w_head3.T)
    rgb_h, den_h = [], []
    for h in range(nsplit):
        xyz_h = lax.dynamic_slice_in_dim(positions, h * nh, nh, 0).T
        enc_flat = _make_encode(nh)(xyz_h.reshape(-1), resf, table_flat)
        enc_t = (enc_flat
                 .reshape(2 * _N_LEVELS // 8, nh // 128, 8, 128)
                 .transpose(0, 2, 1, 3)
                 .reshape(2 * _N_LEVELS, nh))
        d_h = lax.dynamic_slice_in_dim(dir_t, h * nh, nh, 1)
        r_t, d_t = _make_mlp(nh)(enc_t, d_h, *wts)
        rgb_h.append(r_t)
        den_h.append(d_t)
    rgb_t = jnp.concatenate(rgb_h, axis=1)
    den_t = jnp.concatenate(den_h, axis=1)
    return rgb_t.T, den_t.reshape(n, 1)


# speedup vs baseline: 1.0639x; 1.0639x over previous
import functools

import jax
import jax.numpy as jnp
import numpy as np
from jax import lax
from jax.experimental import pallas as pl
from jax.experimental.pallas import tpu as pltpu
from jax.experimental.pallas import tpu_sc as plsc

_N_LEVELS = 16
_TABLE_SIZE = 2 ** 19
_MASK = _TABLE_SIZE - 1
_BASE_RES = 16
_MAX_RES = 4096
_SCALE = np.exp((np.log(_MAX_RES) - np.log(_BASE_RES)) / (_N_LEVELS - 1))
_RES = [int(np.floor(_BASE_RES * _SCALE ** l)) for l in range(_N_LEVELS)]
_P1 = np.uint32(2654435761).astype(np.int32)
_P2 = np.int32(805459861)

_NW = 32
_C = 128
_LANES = 16


def _vperm(v, idx):
    return lax.gather(
        v, idx[:, None],
        lax.GatherDimensionNumbers(offset_dims=(), collapsed_slice_dims=(0,),
                                   start_index_map=(0,)),
        slice_sizes=(1,), mode=lax.GatherScatterMode.PROMISE_IN_BOUNDS)


def _enc_body(xyz_hbm, resf_hbm, table_hbm, out_hbm,
              pos_v, acc_v, resf_v, idx_v, w_v, rows_v, spm_v,
              sem0, sem1, sem2):
    n = out_hbm.shape[0] // (2 * _N_LEVELS)
    b = n // _NW
    nch = b // _C
    sid = lax.axis_index("s")
    wid = sid * 2 + lax.axis_index("c")
    base = wid * b
    sl = _TABLE_SIZE // 16
    for k in range(3):
        pltpu.sync_copy(xyz_hbm.at[pl.ds(k * n + base, b)], pos_v.at[k])
    pltpu.sync_copy(resf_hbm, resf_v)
    resvec = resf_v[pl.ds(0, _LANES)]
    sems = (sem0, sem1)

    def stage(l):
        pltpu.async_copy(
            table_hbm.at[pl.ds(l * _TABLE_SIZE + sid * sl, sl)],
            spm_v.at[l & 1, pl.ds(sid * sl, sl)], sem2)

    def stage_wait():
        pltpu.make_async_copy(table_hbm.at[pl.ds(0, sl)],
                              spm_v.at[0, pl.ds(0, sl)], sem2).wait()

    stage(0)
    stage_wait()
    plsc.subcore_barrier()

    def level_body(l, _):
        hs = _vperm(resvec, jnp.broadcast_to(l, (_LANES,))) * 0.5
        arow = (l & 3) * 2
        tsrc = spm_v.at[l & 1]

        def idxgen_fire(c, buf):
            for i in range(_C // _LANES):
                o = c * _C + i * _LANES
                px = pos_v[0, pl.ds(o, _LANES)]
                py = pos_v[1, pl.ds(o, _LANES)]
                pz = pos_v[2, pl.ds(o, _LANES)]
                posx = px * hs + hs
                posy = py * hs + hs
                posz = pz * hs + hs
                ix = posx.astype(jnp.int32)
                iy = posy.astype(jnp.int32)
                iz = posz.astype(jnp.int32)
                fx = posx - ix.astype(jnp.float32)
                fy = posy - iy.astype(jnp.float32)
                fz = posz - iz.astype(jnp.float32)
                hx = (ix, ix + 1)
                t1 = iy * _P1
                t2 = iz * _P2
                hy = (t1, t1 + _P1)
                hz = (t2, t2 + _P2)
                wx = (1.0 - fx, fx)
                wy = (1.0 - fy, fy)
                wz = (1.0 - fz, fz)
                wyz = {(dy, dz): wy[dy] * wz[dz]
                       for dy in (0, 1) for dz in (0, 1)}
                for dx in (0, 1):
                    for dy in (0, 1):
                        for dz in (0, 1):
                            ci = dx * 4 + dy * 2 + dz
                            idx_v[buf,
                                  pl.ds(ci * _C + i * _LANES, _LANES)] = (
                                ((hx[dx] ^ hy[dy]) ^ hz[dz]) & _MASK)
                            w_v[buf, ci, pl.ds(i * _LANES, _LANES)] = (
                                wx[dx] * wyz[(dy, dz)])
            pltpu.async_copy(tsrc.at[idx_v.at[buf]],
                             rows_v.at[buf], sems[buf])

        def drain(buf):
            pltpu.make_async_copy(tsrc.at[idx_v.at[buf]],
                                  rows_v.at[buf], sems[buf]).wait()

        def accum(c, buf):
            for i in range(_C // _LANES):
                acc = None
                for ci in range(8):
                    u = rows_v[buf, pl.ds(ci * _C + i * _LANES, _LANES)]
                    r = plsc.bitcast(u, jnp.bfloat16)
                    wv = w_v[buf, ci, pl.ds(i * _LANES, _LANES)]
                    wd = plsc.pack(wv, wv,
                                   format=plsc.PackFormat.INTERLEAVED)
                    acc = r * wd if acc is None else acc + r * wd
                f0a, f1a = plsc.unpack(acc,
                                       format=plsc.PackFormat.INTERLEAVED)
                o = c * (8 * _C) + arow * 128 + i * _LANES
                acc_v[pl.ds(o, _LANES)] = f0a
                acc_v[pl.ds(o + 128, _LANES)] = f1a

        idxgen_fire(0, 0)
        stage(jnp.minimum(l + 1, _N_LEVELS - 1))

        def pair(p, _):
            cc = p * 2
            idxgen_fire(cc + 1, 1)
            drain(0)
            accum(cc, 0)

            @pl.when(cc + 2 < nch)
            def _():
                idxgen_fire(cc + 2, 0)

            drain(1)
            accum(cc + 1, 1)
            return ()

        lax.fori_loop(0, nch // 2, pair, (), unroll=False)

        @pl.when((l & 3) == 3)
        def _():
            g = lax.shift_right_logical(l, 2)
            pltpu.sync_copy(acc_v,
                            out_hbm.at[pl.ds(g * (8 * n) + base * 8, 8 * b)])
        stage_wait()
        plsc.subcore_barrier()
        return ()

    lax.fori_loop(0, _N_LEVELS, level_body, (), unroll=False)


def _make_encode(n, interpret=False):
    b = n // _NW
    mesh = plsc.VectorSubcoreMesh(core_axis_name="c", subcore_axis_name="s",
                                  num_cores=2, num_subcores=16)
    return functools.partial(
        pl.kernel, _enc_body,
        out_type=jax.ShapeDtypeStruct((2 * _N_LEVELS * n,), jnp.float32),
        mesh=mesh,
        compiler_params=pltpu.CompilerParams(needs_layout_passes=False,
                                             use_tc_tiling_on_sc=False),
        scratch_types=[
            pltpu.VMEM((3, b), jnp.float32),
            pltpu.VMEM((8 * b,), jnp.float32),
            pltpu.VMEM((128,), jnp.float32),
            pltpu.VMEM((2, 8 * _C), jnp.int32),
            pltpu.VMEM((2, 8, _C), jnp.float32),
            pltpu.VMEM((2, 8 * _C), jnp.uint32),
            pltpu.VMEM_SHARED((2, _TABLE_SIZE), jnp.uint32),
            pltpu.SemaphoreType.DMA,
            pltpu.SemaphoreType.DMA,
            pltpu.SemaphoreType.DMA,
        ],
        interpret=interpret,
    )()


def _pack_body(x_ref, o_ref):
    x = x_ref[...]
    u = lax.bitcast_convert_type(x, jnp.uint32)
    rnd = (u + jnp.uint32(0x7FFF) + ((u >> 16) & jnp.uint32(1))) >> 16
    k = o_ref.shape[0]
    r3 = rnd.reshape(k, 2, 128)
    o_ref[...] = r3[:, 0, :] | (r3[:, 1, :] << 16)


def _make_pack(rows, bk=4096):
    grid = rows // bk
    return pl.pallas_call(
        _pack_body,
        grid=(grid,),
        in_specs=[pl.BlockSpec((bk, 128), lambda i: (i, 0))],
        out_specs=pl.BlockSpec((bk // 2, 128), lambda i: (i, 0)),
        out_shape=jax.ShapeDtypeStruct((rows // 2, 128), jnp.uint32),
    )


def _mm(a, bm):
    return lax.dot_general(a, bm, (((1,), (0,)), ((), ())),
                           preferred_element_type=jnp.float32)


def _mlp_body(enc_ref, dir_ref, w1_ref, w2_ref, h1_ref, h2_ref, h3_ref,
              rgb_ref, den_ref):
    e = enc_ref[...]
    h = jnp.maximum(_mm(w1_ref[...], e), 0.0)
    bo = _mm(w2_ref[...], h)
    den_ref[...] = jnp.exp(bo[0:1, :] - 1.0)
    geo = bo[1:, :]

    d = dir_ref[...]
    dd = ((d + 1.0) * 0.5) * 2.0 - 1.0
    x, y, z = dd[0:1, :], dd[1:2, :], dd[2:3, :]
    x2, y2, z2 = x * x, y * y, z * z
    xy, yz, xz = x * y, y * z, x * z
    sh = [
        jnp.full_like(x, 0.28209479177387814),
        -0.48860251190291987 * y,
        0.48860251190291987 * z,
        -0.48860251190291987 * x,
        1.0925484305920792 * xy,
        -1.0925484305920792 * yz,
        0.94617469575755997 * z2 - 0.31539156525252005,
        -1.0925484305920792 * xz,
        0.54627421529603959 * (x2 - y2),
        0.59004358992664352 * y * (-3.0 * x2 + y2),
        2.8906114426405538 * xy * z,
        0.45704579946446572 * y * (1.0 - 5.0 * z2),
        0.3731763325901154 * z * (5.0 * z2 - 3.0),
        0.45704579946446572 * x * (1.0 - 5.0 * z2),
        1.4453057213202769 * z * (x2 - y2),
        0.59004358992664352 * x * (-x2 + 3.0 * y2),
    ]
    h1 = h1_ref[...]
    hh = _mm(h1[:, 16:], geo)
    for j in range(16):
        hh = hh + h1[:, j:j + 1] * sh[j]
    hh = jnp.maximum(hh, 0.0)
    hh = jnp.maximum(_mm(h2_ref[...], hh), 0.0)
    rgb_ref[...] = jax.nn.sigmoid(_mm(h3_ref[...], hh))


def _make_mlp(n, bn=4096, interpret=False):
    grid = n // bn
    full = lambda shape: pl.BlockSpec(shape, lambda i: (0, 0))
    return pl.pallas_call(
        _mlp_body,
        grid=(grid,),
        in_specs=[
            pl.BlockSpec((2 * _N_LEVELS, bn), lambda i: (0, i)),
            pl.BlockSpec((3, bn), lambda i: (0, i)),
            full((64, 2 * _N_LEVELS)),
            full((16, 64)),
            full((64, 31)),
            full((64, 64)),
            full((3, 64)),
        ],
        out_specs=[
            pl.BlockSpec((3, bn), lambda i: (0, i)),
            pl.BlockSpec((1, bn), lambda i: (0, i)),
        ],
        out_shape=[
            jax.ShapeDtypeStruct((3, n), jnp.float32),
            jax.ShapeDtypeStruct((1, n), jnp.float32),
        ],
        interpret=interpret,
    )


def kernel(positions, directions, hash_table,
           w_base1, w_base2, w_head1, w_head2, w_head3):
    n = positions.shape[0]
    xyz = positions.T.reshape(-1)
    nrows = _N_LEVELS * (_TABLE_SIZE // 128) * 2
    table_view = (hash_table
                  .reshape(_N_LEVELS, _TABLE_SIZE // 128, 128, 2)
                  .transpose(0, 1, 3, 2)
                  .reshape(nrows, 128))
    table_flat = _make_pack(nrows)(table_view).reshape(-1)
    resf = jnp.asarray(np.array(_RES + [0] * (128 - _N_LEVELS),
                                dtype=np.float32))
    nsplit = 2
    nh = n // nsplit
    dir_t = directions.T
    wts = (w_base1.T, w_base2.T, w_head1.T, w_head2.T, w_head3.T)
    rgb_h, den_h = [], []
    for h in range(nsplit):
        xyz_h = lax.dynamic_slice_in_dim(positions, h * nh, nh, 0).T
        enc_flat = _make_encode(nh)(xyz_h.reshape(-1), resf, table_flat)
        enc_t = (enc_flat
                 .reshape(2 * _N_LEVELS // 8, nh // 128, 8, 128)
                 .transpose(0, 2, 1, 3)
                 .reshape(2 * _N_LEVELS, nh))
        d_h = lax.dynamic_slice_in_dim(dir_t, h * nh, nh, 1)
        r_t, d_t = _make_mlp(nh)(enc_t, d_h, *wts)
        rgb_h.append(r_t)
        den_h.append(d_t)
    rgb_t = jnp.concatenate(rgb_h, axis=1)
    den_t = jnp.concatenate(den_h, axis=1)
    return rgb_t.T, den_t.reshape(n, 1)
